# Initial kernel scaffold; baseline (speedup 1.0000x reference)
import jax
import jax.numpy as jnp
from jax.experimental import pallas as pl


def kernel(x, edge_index, edge_attr, atom_tables, bond_tables, W1, b1, bn1_g, bn1_b, W2, b2, eps, obn_g, obn_b):
    raise NotImplementedError("write your pallas kernel here")


# speedup vs baseline: 3.5359x; 3.5359x over previous
import functools

import jax
import jax.numpy as jnp
from jax import lax
from jax.experimental import pallas as pl
from jax.experimental.pallas import tpu as pltpu
from jax.experimental.pallas import tpu_sc as plsc

N = 10000
E = 320000
D = 128
L = 3
NF = 9
BF = 3
AV = 120
BV = 6

NC = 2
NS = 16
NW = NC * NS

N_PAD = 10240
NODES_W = N_PAD // NW
PCH = 80
N_CHUNKS = NODES_W // PCH

E_W = E // NW
ECH = 80
E_CHUNKS = E_W // ECH
AGG_PAD = 10240
ROWS_T = AGG_PAD // NS

CB = BV * BV * BV

def _wid():
    return lax.axis_index("c") * NS + lax.axis_index("s")


def _mesh():
    return plsc.VectorSubcoreMesh(core_axis_name="c", subcore_axis_name="s",
                                  num_cores=NC, num_subcores=NS)


def _atom_kernel(ax_pad, tbl):
    k = pl.kernel(
        _atom_body,
        out_type=jax.ShapeDtypeStruct((N_PAD, D), jnp.float32),
        mesh=_mesh(),
        scratch_types=[
            pltpu.VMEM((NF, PCH), jnp.int32),
            pltpu.VMEM((NF, PCH, D), jnp.float32),
            pltpu.VMEM((PCH, D), jnp.float32),
            pltpu.SemaphoreType.DMA,
        ],
    )
    return k(ax_pad, tbl)


def _atom_body(ax_hbm, tbl_hbm, out_hbm, idx_v, rows_v, out_v, sem):
    wid = _wid()

    def chunk(k, carry):
        nb = wid * NODES_W + k * PCH
        for f in range(NF):
            pltpu.sync_copy(ax_hbm.at[pl.ds(f * N_PAD + nb, PCH)], idx_v.at[f])
        cps = [pltpu.async_copy(tbl_hbm.at[idx_v.at[f]], rows_v.at[f], sem)
               for f in range(NF)]
        for cp in cps:
            cp.wait()

        def node(e, carry2):
            for g in range(D // 16):
                sl = pl.ds(g * 16, 16)
                acc = rows_v[0, e, sl]
                for f in range(1, NF):
                    acc = acc + rows_v[f, e, sl]
                out_v[e, sl] = acc
            return carry2

        lax.fori_loop(0, PCH, node, 0)
        pltpu.sync_copy(out_v, out_hbm.at[pl.ds(nb, PCH)])
        return carry

    lax.fori_loop(0, N_CHUNKS, chunk, 0)


def _edge_kernel(h, src, dst, ci, cbl):
    k = pl.kernel(
        _edge_body,
        out_type=jax.ShapeDtypeStruct((NC, AGG_PAD, D), jnp.float32),
        mesh=_mesh(),
        scratch_types=[
            pltpu.VMEM((CB * D,), jnp.float32),
            pltpu.VMEM((ECH,), jnp.int32),
            pltpu.VMEM((ECH,), jnp.int32),
            pltpu.VMEM((ECH,), jnp.int32),
            pltpu.VMEM((ECH, D), jnp.float32),
            pltpu.VMEM_SHARED((AGG_PAD, D), jnp.float32),
            pltpu.SemaphoreType.DMA,
        ],
    )
    return k(h, src, dst, ci, cbl)


def _edge_body(h_hbm, si_hbm, di_hbm, ci_hbm, cb_hbm, out_hbm,
               cb_v, si_v, di_v, ci_v, rows_v, agg_sh, sem):
    c = lax.axis_index("c")
    s = lax.axis_index("s")
    wid = c * NS + s

    pltpu.sync_copy(cb_hbm, cb_v)

    def zrow(j, carry):
        for g in range(D // 16):
            rows_v[j, pl.ds(g * 16, 16)] = jnp.zeros((16,), jnp.float32)
        return carry

    lax.fori_loop(0, ECH, zrow, 0)
    tbase = s * ROWS_T
    for j in range(ROWS_T // ECH):
        pltpu.sync_copy(rows_v, agg_sh.at[pl.ds(tbase + j * ECH, ECH)])
    plsc.subcore_barrier()

    iota16 = lax.iota(jnp.int32, 16)

    def chunk(k, carry):
        eb = wid * E_W + k * ECH
        pltpu.sync_copy(si_hbm.at[pl.ds(eb, ECH)], si_v)
        pltpu.sync_copy(di_hbm.at[pl.ds(eb, ECH)], di_v)
        pltpu.sync_copy(ci_hbm.at[pl.ds(eb, ECH)], ci_v)
        pltpu.async_copy(h_hbm.at[si_v], rows_v, sem).wait()

        def egroup(j, carry2):
            civ = ci_v[pl.ds(j * 16, 16)] * D
            for i in range(16):
                e = j * 16 + i
                rowb = civ[i]
                for g in range(D // 16):
                    sl = pl.ds(g * 16, 16)
                    ee = cb_v[pl.ds(rowb + g * 16, 16)]
                    rows_v[e, sl] = jnp.maximum(rows_v[e, sl] + ee, 0.0)
            return carry2

        lax.fori_loop(0, ECH // 16, egroup, 0)
        pltpu.sync_copy(rows_v, agg_sh.at[di_v], add=True)
        return carry

    lax.fori_loop(0, E_CHUNKS, chunk, 0)
    plsc.subcore_barrier()
    pltpu.sync_copy(agg_sh.at[pl.ds(tbase, ROWS_T)],
                    out_hbm.at[c, pl.ds(tbase, ROWS_T)])


BR = 1000
NB = N // BR


def _tc1_body(epsm_ref, h_ref, a0_ref, a1_ref, w_ref, b_ref,
              y_ref, st_ref, acc_ref):
    t = epsm_ref[0, 0] * h_ref[...] + a0_ref[...] + a1_ref[...]
    y = jnp.dot(t, w_ref[...], preferred_element_type=jnp.float32) + b_ref[...]
    y_ref[...] = y
    i = pl.program_id(0)
    contrib = jnp.stack([jnp.sum(y, axis=0), jnp.sum(y * y, axis=0)])
    prev = jnp.where(i == 0, jnp.zeros_like(contrib), acc_ref[...])
    acc_ref[...] = prev + contrib

    @pl.when(i == NB - 1)
    def _():
        st_ref[...] = acc_ref[...]


def _tc2_body(y1_ref, st_ref, g_ref, bb_ref, w_ref, b_ref,
              y_ref, st2_ref, acc_ref):
    m = st_ref[0] * (1.0 / N)
    v = st_ref[1] * (1.0 / N) - m * m
    z = (y1_ref[...] - m) * lax.rsqrt(v + 1e-5) * g_ref[...] + bb_ref[...]
    z = jnp.maximum(z, 0.0)
    y = jnp.dot(z, w_ref[...], preferred_element_type=jnp.float32) + b_ref[...]
    y_ref[...] = y
    i = pl.program_id(0)
    contrib = jnp.stack([jnp.sum(y, axis=0), jnp.sum(y * y, axis=0)])
    prev = jnp.where(i == 0, jnp.zeros_like(contrib), acc_ref[...])
    acc_ref[...] = prev + contrib

    @pl.when(i == NB - 1)
    def _():
        st2_ref[...] = acc_ref[...]


def _tc3_body(relu, y2_ref, st_ref, g_ref, bb_ref, h_ref):
    m = st_ref[0] * (1.0 / N)
    v = st_ref[1] * (1.0 / N) - m * m
    h = (y2_ref[...] - m) * lax.rsqrt(v + 1e-5) * g_ref[...] + bb_ref[...]
    if relu:
        h = jnp.maximum(h, 0.0)
    h_ref[...] = h


def _row_spec(cols):
    return pl.BlockSpec((BR, cols), lambda i: (i, 0))


def _full_spec(r, cols):
    return pl.BlockSpec((r, cols), lambda i: (0, 0))


def _tc1(epsm, h, a0, a1, w1, b1):
    return pl.pallas_call(
        _tc1_body,
        grid=(NB,),
        in_specs=[
            pl.BlockSpec(memory_space=pltpu.SMEM),
            _row_spec(D), _row_spec(D), _row_spec(D),
            _full_spec(D, 2 * D), _full_spec(1, 2 * D),
        ],
        out_specs=[_row_spec(2 * D), _full_spec(2, 2 * D)],
        out_shape=[jax.ShapeDtypeStruct((N, 2 * D), jnp.float32),
                   jax.ShapeDtypeStruct((2, 2 * D), jnp.float32)],
        scratch_shapes=[pltpu.VMEM((2, 2 * D), jnp.float32)],
    )(epsm, h, a0, a1, w1, b1)


def _tc2(y1, st1, g1, bb1, w2, b2):
    return pl.pallas_call(
        _tc2_body,
        grid=(NB,),
        in_specs=[
            _row_spec(2 * D), _full_spec(2, 2 * D),
            _full_spec(1, 2 * D), _full_spec(1, 2 * D),
            _full_spec(2 * D, D), _full_spec(1, D),
        ],
        out_specs=[_row_spec(D), _full_spec(2, D)],
        out_shape=[jax.ShapeDtypeStruct((N, D), jnp.float32),
                   jax.ShapeDtypeStruct((2, D), jnp.float32)],
        scratch_shapes=[pltpu.VMEM((2, D), jnp.float32)],
    )(y1, st1, g1, bb1, w2, b2)


def _tc3(y2, st2, g, bb, relu):
    return pl.pallas_call(
        functools.partial(_tc3_body, relu),
        grid=(NB,),
        in_specs=[
            _row_spec(D), _full_spec(2, D),
            _full_spec(1, D), _full_spec(1, D),
        ],
        out_specs=_row_spec(D),
        out_shape=jax.ShapeDtypeStruct((N, D), jnp.float32),
    )(y2, st2, g, bb)


def kernel(x, edge_index, edge_attr, atom_tables, bond_tables,
           W1, b1, bn1_g, bn1_b, W2, b2, eps, obn_g, obn_b):
    x = x.astype(jnp.int32)
    edge_index = edge_index.astype(jnp.int32)
    edge_attr = edge_attr.astype(jnp.int32)

    ax = x.T + (jnp.arange(NF, dtype=jnp.int32) * AV)[:, None]
    ax_pad = jnp.pad(ax, ((0, 0), (0, N_PAD - N))).reshape(NF * N_PAD)
    tbl = atom_tables.reshape(NF * AV, D)

    bt = bond_tables
    cb = (bt[:, 0][:, :, None, None, :] + bt[:, 1][:, None, :, None, :]
          + bt[:, 2][:, None, None, :, :]).reshape(L, CB * D)

    src = edge_index[0]
    dst = edge_index[1]
    ci = (edge_attr[:, 0] * BV + edge_attr[:, 1]) * BV + edge_attr[:, 2]

    h = _atom_kernel(ax_pad, tbl)[:N]
    for layer in range(L):
        agg2 = _edge_kernel(h, src, dst, ci, cb[layer])
        epsm = (1.0 + eps[layer]).reshape(1, 1)
        y1, st1 = _tc1(epsm, h, agg2[0, :N], agg2[1, :N], W1[layer],
                       b1[layer].reshape(1, 2 * D))
        y2, st2 = _tc2(y1, st1, bn1_g[layer].reshape(1, 2 * D),
                       bn1_b[layer].reshape(1, 2 * D), W2[layer],
                       b2[layer].reshape(1, D))
        h = _tc3(y2, st2, obn_g[layer].reshape(1, D),
                 obn_b[layer].reshape(1, D), relu=(layer < L - 1))
    return h


# speedup vs baseline: 3.8240x; 1.0815x over previous
import functools

import jax
import jax.numpy as jnp
from jax import lax
from jax.experimental import pallas as pl
from jax.experimental.pallas import tpu as pltpu
from jax.experimental.pallas import tpu_sc as plsc

N = 10000
E = 320000
D = 128
L = 3
NF = 9
BF = 3
AV = 120
BV = 6

NC = 2
NS = 16
NW = NC * NS

N_PAD = 10240
NODES_W = N_PAD // NW
PCH = 80
N_CHUNKS = NODES_W // PCH

SUB = 80
EB = 160
NBLK_W = 63
E_W_PAD = NBLK_W * EB
E_PAD = E_W_PAD * NW
NBLK = E_PAD // EB
IDXB = 6 * SUB
AGG_PAD = 10112
ROWS_T = AGG_PAD // NS
DUMMY_DST = AGG_PAD - 8

CB = BV * BV * BV

def _wid():
    return lax.axis_index("c") * NS + lax.axis_index("s")


def _mesh():
    return plsc.VectorSubcoreMesh(core_axis_name="c", subcore_axis_name="s",
                                  num_cores=NC, num_subcores=NS)


def _atom_kernel(ax_pad, tbl):
    k = pl.kernel(
        _atom_body,
        out_type=jax.ShapeDtypeStruct((N_PAD, D), jnp.float32),
        mesh=_mesh(),
        scratch_types=[
            pltpu.VMEM((NF, PCH), jnp.int32),
            pltpu.VMEM((NF, PCH, D), jnp.float32),
            pltpu.VMEM((PCH, D), jnp.float32),
            pltpu.SemaphoreType.DMA,
        ],
    )
    return k(ax_pad, tbl)


def _atom_body(ax_hbm, tbl_hbm, out_hbm, idx_v, rows_v, out_v, sem):
    wid = _wid()

    def chunk(k, carry):
        nb = wid * NODES_W + k * PCH
        for f in range(NF):
            pltpu.sync_copy(ax_hbm.at[pl.ds(f * N_PAD + nb, PCH)], idx_v.at[f])
        cps = [pltpu.async_copy(tbl_hbm.at[idx_v.at[f]], rows_v.at[f], sem)
               for f in range(NF)]
        for cp in cps:
            cp.wait()

        def node(e, carry2):
            for g in range(D // 16):
                sl = pl.ds(g * 16, 16)
                acc = rows_v[0, e, sl]
                for f in range(1, NF):
                    acc = acc + rows_v[f, e, sl]
                out_v[e, sl] = acc
            return carry2

        lax.fori_loop(0, PCH, node, 0)
        pltpu.sync_copy(out_v, out_hbm.at[pl.ds(nb, PCH)])
        return carry

    lax.fori_loop(0, N_CHUNKS, chunk, 0)


def _edge_kernel(h, idx_all, cbl):
    k = pl.kernel(
        _edge_body,
        out_type=jax.ShapeDtypeStruct((NC, AGG_PAD, D), jnp.float32),
        mesh=_mesh(),
        scratch_types=[
            pltpu.VMEM((CB * D,), jnp.float32),
            pltpu.VMEM((IDXB,), jnp.int32),
            pltpu.VMEM((2, SUB), jnp.int32),
            pltpu.VMEM((EB, D), jnp.float32),
            pltpu.VMEM_SHARED((AGG_PAD, D), jnp.float32),
            pltpu.SemaphoreType.DMA,
        ],
    )
    return k(h, idx_all, cbl)


def _edge_body(h_hbm, idx_hbm, cb_hbm, out_hbm,
               cb_v, idx_v, di2_v, rows_v, agg_sh, sem):
    c = lax.axis_index("c")
    s = lax.axis_index("s")
    wid = c * NS + s

    pltpu.sync_copy(cb_hbm, cb_v)

    def zrow(j, carry):
        for g in range(D // 16):
            rows_v[j, pl.ds(g * 16, 16)] = jnp.zeros((16,), jnp.float32)
        return carry

    lax.fori_loop(0, EB, zrow, 0)
    tbase = s * ROWS_T
    for j in range(ROWS_T // EB):
        pltpu.sync_copy(rows_v, agg_sh.at[pl.ds(tbase + j * EB, EB)])
    rem = ROWS_T - (ROWS_T // EB) * EB
    if rem:
        pltpu.sync_copy(rows_v.at[pl.ds(0, rem)],
                        agg_sh.at[pl.ds(tbase + (ROWS_T // EB) * EB, rem)])
    plsc.subcore_barrier()

    def block(k, carry):
        off = (wid * NBLK_W + k) * IDXB
        pltpu.sync_copy(idx_hbm.at[pl.ds(off, IDXB)], idx_v)
        g0 = pltpu.async_copy(h_hbm.at[idx_v.at[pl.ds(0, SUB)]],
                              rows_v.at[pl.ds(0, SUB)], sem)
        g1 = pltpu.async_copy(h_hbm.at[idx_v.at[pl.ds(SUB, SUB)]],
                              rows_v.at[pl.ds(SUB, SUB)], sem)
        for jj in range(EB // 16):
            di2_v[jj // (SUB // 16), pl.ds((jj % (SUB // 16)) * 16, 16)] = (
                idx_v[pl.ds(4 * SUB + jj * 16, 16)])
        g0.wait()
        g1.wait()

        def egroup(j, carry2):
            civ = idx_v[pl.ds(2 * SUB + j * 16, 16)] * D
            for i in range(16):
                e = j * 16 + i
                rowb = civ[i]
                for g in range(D // 16):
                    sl = pl.ds(g * 16, 16)
                    ee = cb_v[pl.ds(rowb + g * 16, 16)]
                    rows_v[e, sl] = jnp.maximum(rows_v[e, sl] + ee, 0.0)
            return carry2

        lax.fori_loop(0, EB // 16, egroup, 0)
        s0 = pltpu.async_copy(rows_v.at[pl.ds(0, SUB)],
                              agg_sh.at[di2_v.at[0]], sem, add=True)
        s1 = pltpu.async_copy(rows_v.at[pl.ds(SUB, SUB)],
                              agg_sh.at[di2_v.at[1]], sem, add=True)
        s0.wait()
        s1.wait()
        return carry

    lax.fori_loop(0, NBLK_W, block, 0)
    plsc.subcore_barrier()
    pltpu.sync_copy(agg_sh.at[pl.ds(tbase, ROWS_T)],
                    out_hbm.at[c, pl.ds(tbase, ROWS_T)])


BR = 1000
NB = N // BR


def _tc1_body(epsm_ref, h_ref, a0_ref, a1_ref, w_ref, b_ref,
              y_ref, st_ref, acc_ref):
    t = epsm_ref[0, 0] * h_ref[...] + a0_ref[...] + a1_ref[...]
    y = jnp.dot(t, w_ref[...], preferred_element_type=jnp.float32) + b_ref[...]
    y_ref[...] = y
    i = pl.program_id(0)
    contrib = jnp.stack([jnp.sum(y, axis=0), jnp.sum(y * y, axis=0)])
    prev = jnp.where(i == 0, jnp.zeros_like(contrib), acc_ref[...])
    acc_ref[...] = prev + contrib

    @pl.when(i == NB - 1)
    def _():
        st_ref[...] = acc_ref[...]


def _tc2_body(y1_ref, st_ref, g_ref, bb_ref, w_ref, b_ref,
              y_ref, st2_ref, acc_ref):
    m = st_ref[0] * (1.0 / N)
    v = st_ref[1] * (1.0 / N) - m * m
    z = (y1_ref[...] - m) * lax.rsqrt(v + 1e-5) * g_ref[...] + bb_ref[...]
    z = jnp.maximum(z, 0.0)
    y = jnp.dot(z, w_ref[...], preferred_element_type=jnp.float32) + b_ref[...]
    y_ref[...] = y
    i = pl.program_id(0)
    contrib = jnp.stack([jnp.sum(y, axis=0), jnp.sum(y * y, axis=0)])
    prev = jnp.where(i == 0, jnp.zeros_like(contrib), acc_ref[...])
    acc_ref[...] = prev + contrib

    @pl.when(i == NB - 1)
    def _():
        st2_ref[...] = acc_ref[...]


def _tc3_body(relu, y2_ref, st_ref, g_ref, bb_ref, h_ref):
    m = st_ref[0] * (1.0 / N)
    v = st_ref[1] * (1.0 / N) - m * m
    h = (y2_ref[...] - m) * lax.rsqrt(v + 1e-5) * g_ref[...] + bb_ref[...]
    if relu:
        h = jnp.maximum(h, 0.0)
    h_ref[...] = h


def _row_spec(cols):
    return pl.BlockSpec((BR, cols), lambda i: (i, 0))


def _full_spec(r, cols):
    return pl.BlockSpec((r, cols), lambda i: (0, 0))


def _tc1(epsm, h, a0, a1, w1, b1):
    return pl.pallas_call(
        _tc1_body,
        grid=(NB,),
        in_specs=[
            pl.BlockSpec(memory_space=pltpu.SMEM),
            _row_spec(D), _row_spec(D), _row_spec(D),
            _full_spec(D, 2 * D), _full_spec(1, 2 * D),
        ],
        out_specs=[_row_spec(2 * D), _full_spec(2, 2 * D)],
        out_shape=[jax.ShapeDtypeStruct((N, 2 * D), jnp.float32),
                   jax.ShapeDtypeStruct((2, 2 * D), jnp.float32)],
        scratch_shapes=[pltpu.VMEM((2, 2 * D), jnp.float32)],
    )(epsm, h, a0, a1, w1, b1)


def _tc2(y1, st1, g1, bb1, w2, b2):
    return pl.pallas_call(
        _tc2_body,
        grid=(NB,),
        in_specs=[
            _row_spec(2 * D), _full_spec(2, 2 * D),
            _full_spec(1, 2 * D), _full_spec(1, 2 * D),
            _full_spec(2 * D, D), _full_spec(1, D),
        ],
        out_specs=[_row_spec(D), _full_spec(2, D)],
        out_shape=[jax.ShapeDtypeStruct((N, D), jnp.float32),
                   jax.ShapeDtypeStruct((2, D), jnp.float32)],
        scratch_shapes=[pltpu.VMEM((2, D), jnp.float32)],
    )(y1, st1, g1, bb1, w2, b2)


def _tc3(y2, st2, g, bb, relu):
    return pl.pallas_call(
        functools.partial(_tc3_body, relu),
        grid=(NB,),
        in_specs=[
            _row_spec(D), _full_spec(2, D),
            _full_spec(1, D), _full_spec(1, D),
        ],
        out_specs=_row_spec(D),
        out_shape=jax.ShapeDtypeStruct((N, D), jnp.float32),
    )(y2, st2, g, bb)


def kernel(x, edge_index, edge_attr, atom_tables, bond_tables,
           W1, b1, bn1_g, bn1_b, W2, b2, eps, obn_g, obn_b):
    x = x.astype(jnp.int32)
    edge_index = edge_index.astype(jnp.int32)
    edge_attr = edge_attr.astype(jnp.int32)

    ax = x.T + (jnp.arange(NF, dtype=jnp.int32) * AV)[:, None]
    ax_pad = jnp.pad(ax, ((0, 0), (0, N_PAD - N))).reshape(NF * N_PAD)
    tbl = atom_tables.reshape(NF * AV, D)

    bt = bond_tables
    cb = (bt[:, 0][:, :, None, None, :] + bt[:, 1][:, None, :, None, :]
          + bt[:, 2][:, None, None, :, :]).reshape(L, CB * D)

    src = edge_index[0]
    dst = edge_index[1]
    ci = (edge_attr[:, 0] * BV + edge_attr[:, 1]) * BV + edge_attr[:, 2]

    npad = E_PAD - E
    sp = jnp.concatenate([src, jnp.zeros((npad,), jnp.int32)]).reshape(
        NBLK, 2, SUB)
    cp = jnp.concatenate([ci, jnp.zeros((npad,), jnp.int32)]).reshape(
        NBLK, 2, SUB)
    dp = jnp.concatenate([dst, jnp.full((npad,), DUMMY_DST, jnp.int32)]
                         ).reshape(NBLK, 2, SUB)
    idx_all = jnp.concatenate([sp, cp, dp], axis=1).reshape(NBLK * IDXB)

    h = _atom_kernel(ax_pad, tbl)[:N]
    for layer in range(L):
        agg2 = _edge_kernel(h, idx_all, cb[layer])
        epsm = (1.0 + eps[layer]).reshape(1, 1)
        y1, st1 = _tc1(epsm, h, agg2[0, :N], agg2[1, :N], W1[layer],
                       b1[layer].reshape(1, 2 * D))
        y2, st2 = _tc2(y1, st1, bn1_g[layer].reshape(1, 2 * D),
                       bn1_b[layer].reshape(1, 2 * D), W2[layer],
                       b2[layer].reshape(1, D))
        h = _tc3(y2, st2, obn_g[layer].reshape(1, D),
                 obn_b[layer].reshape(1, D), relu=(layer < L - 1))
    return h


# speedup vs baseline: 5.7301x; 1.4985x over previous
import functools

import jax
import jax.numpy as jnp
from jax import lax
from jax.experimental import pallas as pl
from jax.experimental.pallas import tpu as pltpu
from jax.experimental.pallas import tpu_sc as plsc

N = 10000
E = 320000
D = 128
L = 3
NF = 9
BF = 3
AV = 120
BV = 6

NC = 2
NS = 16
NW = NC * NS

N_PAD = 10240
NODES_W = N_PAD // NW
PCH = 80
N_CHUNKS = NODES_W // PCH

EB = 80
NBLK_W = 126
E_W_PAD = NBLK_W * EB
E_PAD = E_W_PAD * NW
NBLK = E_PAD // EB
AGG_PAD = 10112
ROWS_T = AGG_PAD // NS
DUMMY_DST = AGG_PAD - 8

CB = BV * BV * BV

def _wid():
    return lax.axis_index("c") * NS + lax.axis_index("s")


def _mesh():
    return plsc.VectorSubcoreMesh(core_axis_name="c", subcore_axis_name="s",
                                  num_cores=NC, num_subcores=NS)


def _atom_kernel(ax_pad, tbl):
    k = pl.kernel(
        _atom_body,
        out_type=jax.ShapeDtypeStruct((N_PAD, D), jnp.float32),
        mesh=_mesh(),
        scratch_types=[
            pltpu.VMEM((NF, PCH), jnp.int32),
            pltpu.VMEM((NF, PCH, D), jnp.float32),
            pltpu.VMEM((PCH, D), jnp.float32),
            pltpu.SemaphoreType.DMA,
        ],
    )
    return k(ax_pad, tbl)


def _atom_body(ax_hbm, tbl_hbm, out_hbm, idx_v, rows_v, out_v, sem):
    wid = _wid()

    def chunk(k, carry):
        nb = wid * NODES_W + k * PCH
        for f in range(NF):
            pltpu.sync_copy(ax_hbm.at[pl.ds(f * N_PAD + nb, PCH)], idx_v.at[f])
        cps = [pltpu.async_copy(tbl_hbm.at[idx_v.at[f]], rows_v.at[f], sem)
               for f in range(NF)]
        for cp in cps:
            cp.wait()

        def node(e, carry2):
            for g in range(D // 16):
                sl = pl.ds(g * 16, 16)
                acc = rows_v[0, e, sl]
                for f in range(1, NF):
                    acc = acc + rows_v[f, e, sl]
                out_v[e, sl] = acc
            return carry2

        lax.fori_loop(0, PCH, node, 0)
        pltpu.sync_copy(out_v, out_hbm.at[pl.ds(nb, PCH)])
        return carry

    lax.fori_loop(0, N_CHUNKS, chunk, 0)


def _edge_kernel(h, idx3, cbl):
    k = pl.kernel(
        _edge_body,
        out_type=jax.ShapeDtypeStruct((NC, AGG_PAD, D), jnp.float32),
        mesh=_mesh(),
        scratch_types=[
            pltpu.VMEM((2, EB, D), jnp.float32),
            pltpu.VMEM((2, EB, D), jnp.float32),
            pltpu.VMEM((2, 8, EB), jnp.int32),
            pltpu.VMEM((2, EB), jnp.int32),
            pltpu.VMEM_SHARED((AGG_PAD, D), jnp.float32),
            pltpu.SemaphoreType.DMA,
            pltpu.SemaphoreType.DMA,
            pltpu.SemaphoreType.DMA,
            pltpu.SemaphoreType.DMA,
            pltpu.SemaphoreType.DMA,
            pltpu.SemaphoreType.DMA,
        ],
    )
    return k(h, idx3, cbl)


def _edge_body(h_hbm, idx_hbm, cb_hbm, out_hbm,
               rows_v, ee_v, idx_v, di_v, agg_sh,
               isem0, isem1, gsem0, gsem1, ssem0, ssem1):
    c = lax.axis_index("c")
    s = lax.axis_index("s")
    wid = c * NS + s
    isem = (isem0, isem1)
    gsem = (gsem0, gsem1)
    ssem = (ssem0, ssem1)

    def zrow(j, carry):
        for g in range(D // 16):
            rows_v[0, j, pl.ds(g * 16, 16)] = jnp.zeros((16,), jnp.float32)
        return carry

    lax.fori_loop(0, EB, zrow, 0)
    tbase = s * ROWS_T
    for j in range(ROWS_T // EB):
        pltpu.sync_copy(rows_v.at[0], agg_sh.at[pl.ds(tbase + j * EB, EB)])
    rem = ROWS_T - (ROWS_T // EB) * EB
    if rem:
        pltpu.sync_copy(rows_v.at[0, pl.ds(0, rem)],
                        agg_sh.at[pl.ds(tbase + (ROWS_T // EB) * EB, rem)])
    plsc.subcore_barrier()

    bbase = wid * NBLK_W

    def fire_idx(kk, b):
        pltpu.async_copy(idx_hbm.at[bbase + kk], idx_v.at[b], isem[b])

    def drain_idx(b):
        pltpu.make_async_copy(idx_hbm.at[0], idx_v.at[b], isem[b]).wait()

    def fire_gathers(b):
        pltpu.async_copy(h_hbm.at[idx_v.at[b, 0]], rows_v.at[b], gsem[b])
        pltpu.async_copy(cb_hbm.at[idx_v.at[b, 1]], ee_v.at[b], gsem[b])

    def drain_gathers(b):
        pltpu.make_async_copy(h_hbm.at[idx_v.at[b, 0]], rows_v.at[b],
                              gsem[b]).wait()
        pltpu.make_async_copy(cb_hbm.at[idx_v.at[b, 1]], ee_v.at[b],
                              gsem[b]).wait()

    def fire_scatter(b):
        pltpu.async_copy(rows_v.at[b], agg_sh.at[di_v.at[b]], ssem[b],
                         add=True)

    def drain_scatter(b):
        pltpu.make_async_copy(rows_v.at[b], agg_sh.at[di_v.at[b]],
                              ssem[b]).wait()

    fire_idx(0, 0)
    fire_idx(1, 1)
    drain_idx(0)
    fire_gathers(0)

    def superstep(ks, carry):
        for b in range(2):
            kk = 2 * ks + b
            drain_gathers(b)
            for jj in range(EB // 16):
                di_v[b, pl.ds(jj * 16, 16)] = idx_v[b, 2, pl.ds(jj * 16, 16)]

            @plsc.parallel_loop(0, EB, unroll=2)
            def _edge(e):
                hs = [rows_v[b, e, pl.ds(g * 16, 16)] for g in range(D // 16)]
                es = [ee_v[b, e, pl.ds(g * 16, 16)] for g in range(D // 16)]
                for g in range(D // 16):
                    rows_v[b, e, pl.ds(g * 16, 16)] = jnp.maximum(
                        hs[g] + es[g], 0.0)

            fire_scatter(b)

            @pl.when(kk >= 1)
            def _():
                drain_scatter(1 - b)

            @pl.when(kk + 2 < NBLK_W)
            def _():
                fire_idx(kk + 2, b)

            @pl.when(kk + 1 < NBLK_W)
            def _():
                drain_idx(1 - b)
                fire_gathers(1 - b)

        return carry

    lax.fori_loop(0, NBLK_W // 2, superstep, 0)
    drain_scatter(1)
    plsc.subcore_barrier()
    pltpu.sync_copy(agg_sh.at[pl.ds(tbase, ROWS_T)],
                    out_hbm.at[c, pl.ds(tbase, ROWS_T)])


BR = 1000
NB = N // BR


def _tc1_body(epsm_ref, h_ref, a0_ref, a1_ref, w_ref, b_ref,
              y_ref, st_ref, acc_ref):
    t = epsm_ref[0, 0] * h_ref[...] + a0_ref[...] + a1_ref[...]
    y = jnp.dot(t, w_ref[...], preferred_element_type=jnp.float32) + b_ref[...]
    y_ref[...] = y
    i = pl.program_id(0)
    contrib = jnp.stack([jnp.sum(y, axis=0), jnp.sum(y * y, axis=0)])
    prev = jnp.where(i == 0, jnp.zeros_like(contrib), acc_ref[...])
    acc_ref[...] = prev + contrib

    @pl.when(i == NB - 1)
    def _():
        st_ref[...] = acc_ref[...]


def _tc2_body(y1_ref, st_ref, g_ref, bb_ref, w_ref, b_ref,
              y_ref, st2_ref, acc_ref):
    m = st_ref[0] * (1.0 / N)
    v = st_ref[1] * (1.0 / N) - m * m
    z = (y1_ref[...] - m) * lax.rsqrt(v + 1e-5) * g_ref[...] + bb_ref[...]
    z = jnp.maximum(z, 0.0)
    y = jnp.dot(z, w_ref[...], preferred_element_type=jnp.float32) + b_ref[...]
    y_ref[...] = y
    i = pl.program_id(0)
    contrib = jnp.stack([jnp.sum(y, axis=0), jnp.sum(y * y, axis=0)])
    prev = jnp.where(i == 0, jnp.zeros_like(contrib), acc_ref[...])
    acc_ref[...] = prev + contrib

    @pl.when(i == NB - 1)
    def _():
        st2_ref[...] = acc_ref[...]


def _tc3_body(relu, y2_ref, st_ref, g_ref, bb_ref, h_ref):
    m = st_ref[0] * (1.0 / N)
    v = st_ref[1] * (1.0 / N) - m * m
    h = (y2_ref[...] - m) * lax.rsqrt(v + 1e-5) * g_ref[...] + bb_ref[...]
    if relu:
        h = jnp.maximum(h, 0.0)
    h_ref[...] = h


def _row_spec(cols):
    return pl.BlockSpec((BR, cols), lambda i: (i, 0))


def _full_spec(r, cols):
    return pl.BlockSpec((r, cols), lambda i: (0, 0))


def _tc1(epsm, h, a0, a1, w1, b1):
    return pl.pallas_call(
        _tc1_body,
        grid=(NB,),
        in_specs=[
            pl.BlockSpec(memory_space=pltpu.SMEM),
            _row_spec(D), _row_spec(D), _row_spec(D),
            _full_spec(D, 2 * D), _full_spec(1, 2 * D),
        ],
        out_specs=[_row_spec(2 * D), _full_spec(2, 2 * D)],
        out_shape=[jax.ShapeDtypeStruct((N, 2 * D), jnp.float32),
                   jax.ShapeDtypeStruct((2, 2 * D), jnp.float32)],
        scratch_shapes=[pltpu.VMEM((2, 2 * D), jnp.float32)],
    )(epsm, h, a0, a1, w1, b1)


def _tc2(y1, st1, g1, bb1, w2, b2):
    return pl.pallas_call(
        _tc2_body,
        grid=(NB,),
        in_specs=[
            _row_spec(2 * D), _full_spec(2, 2 * D),
            _full_spec(1, 2 * D), _full_spec(1, 2 * D),
            _full_spec(2 * D, D), _full_spec(1, D),
        ],
        out_specs=[_row_spec(D), _full_spec(2, D)],
        out_shape=[jax.ShapeDtypeStruct((N, D), jnp.float32),
                   jax.ShapeDtypeStruct((2, D), jnp.float32)],
        scratch_shapes=[pltpu.VMEM((2, D), jnp.float32)],
    )(y1, st1, g1, bb1, w2, b2)


def _tc3(y2, st2, g, bb, relu):
    return pl.pallas_call(
        functools.partial(_tc3_body, relu),
        grid=(NB,),
        in_specs=[
            _row_spec(D), _full_spec(2, D),
            _full_spec(1, D), _full_spec(1, D),
        ],
        out_specs=_row_spec(D),
        out_shape=jax.ShapeDtypeStruct((N, D), jnp.float32),
    )(y2, st2, g, bb)


def kernel(x, edge_index, edge_attr, atom_tables, bond_tables,
           W1, b1, bn1_g, bn1_b, W2, b2, eps, obn_g, obn_b):
    x = x.astype(jnp.int32)
    edge_index = edge_index.astype(jnp.int32)
    edge_attr = edge_attr.astype(jnp.int32)

    ax = x.T + (jnp.arange(NF, dtype=jnp.int32) * AV)[:, None]
    ax_pad = jnp.pad(ax, ((0, 0), (0, N_PAD - N))).reshape(NF * N_PAD)
    tbl = atom_tables.reshape(NF * AV, D)

    bt = bond_tables
    cb = (bt[:, 0][:, :, None, None, :] + bt[:, 1][:, None, :, None, :]
          + bt[:, 2][:, None, None, :, :]).reshape(L, CB * D)

    src = edge_index[0]
    dst = edge_index[1]
    ci = (edge_attr[:, 0] * BV + edge_attr[:, 1]) * BV + edge_attr[:, 2]

    npad = E_PAD - E
    sp = jnp.concatenate([src, jnp.zeros((npad,), jnp.int32)]).reshape(
        NBLK, 1, EB)
    cp = jnp.concatenate([ci, jnp.zeros((npad,), jnp.int32)]).reshape(
        NBLK, 1, EB)
    dp = jnp.concatenate([dst, jnp.full((npad,), DUMMY_DST, jnp.int32)]
                         ).reshape(NBLK, 1, EB)
    idx3 = jnp.concatenate(
        [sp, cp, dp, jnp.zeros((NBLK, 5, EB), jnp.int32)], axis=1)

    h = _atom_kernel(ax_pad, tbl)[:N]
    for layer in range(L):
        agg2 = _edge_kernel(h, idx3, cb[layer].reshape(CB, D))
        epsm = (1.0 + eps[layer]).reshape(1, 1)
        y1, st1 = _tc1(epsm, h, agg2[0, :N], agg2[1, :N], W1[layer],
                       b1[layer].reshape(1, 2 * D))
        y2, st2 = _tc2(y1, st1, bn1_g[layer].reshape(1, 2 * D),
                       bn1_b[layer].reshape(1, 2 * D), W2[layer],
                       b2[layer].reshape(1, D))
        h = _tc3(y2, st2, obn_g[layer].reshape(1, D),
                 obn_b[layer].reshape(1, D), relu=(layer < L - 1))
    return h


# speedup vs baseline: 5.7597x; 1.0052x over previous
import functools

import jax
import jax.numpy as jnp
from jax import lax
from jax.experimental import pallas as pl
from jax.experimental.pallas import tpu as pltpu
from jax.experimental.pallas import tpu_sc as plsc

N = 10000
E = 320000
D = 128
L = 3
NF = 9
BF = 3
AV = 120
BV = 6

NC = 2
NS = 16
NW = NC * NS

N_PAD = 10240
NODES_W = N_PAD // NW
PCH = 80
N_CHUNKS = NODES_W // PCH

EB = 64
NBLK_W = 158
E_W_PAD = NBLK_W * EB
E_PAD = E_W_PAD * NW
NBLK = E_PAD // EB
AGG_PAD = 10112
ROWS_T = AGG_PAD // NS
DUMMY_DST = AGG_PAD - 8

CB = BV * BV * BV

def _wid():
    return lax.axis_index("c") * NS + lax.axis_index("s")


def _mesh():
    return plsc.VectorSubcoreMesh(core_axis_name="c", subcore_axis_name="s",
                                  num_cores=NC, num_subcores=NS)


def _atom_kernel(ax_pad, tbl):
    k = pl.kernel(
        _atom_body,
        out_type=jax.ShapeDtypeStruct((N_PAD, D), jnp.float32),
        mesh=_mesh(),
        scratch_types=[
            pltpu.VMEM((NF, PCH), jnp.int32),
            pltpu.VMEM((NF, PCH, D), jnp.float32),
            pltpu.VMEM((PCH, D), jnp.float32),
            pltpu.SemaphoreType.DMA,
        ],
    )
    return k(ax_pad, tbl)


def _atom_body(ax_hbm, tbl_hbm, out_hbm, idx_v, rows_v, out_v, sem):
    wid = _wid()

    def chunk(k, carry):
        nb = wid * NODES_W + k * PCH
        for f in range(NF):
            pltpu.sync_copy(ax_hbm.at[pl.ds(f * N_PAD + nb, PCH)], idx_v.at[f])
        cps = [pltpu.async_copy(tbl_hbm.at[idx_v.at[f]], rows_v.at[f], sem)
               for f in range(NF)]
        for cp in cps:
            cp.wait()

        def node(e, carry2):
            for g in range(D // 16):
                sl = pl.ds(g * 16, 16)
                acc = rows_v[0, e, sl]
                for f in range(1, NF):
                    acc = acc + rows_v[f, e, sl]
                out_v[e, sl] = acc
            return carry2

        lax.fori_loop(0, PCH, node, 0)
        pltpu.sync_copy(out_v, out_hbm.at[pl.ds(nb, PCH)])
        return carry

    lax.fori_loop(0, N_CHUNKS, chunk, 0)


def _edge_kernel(h, idx3, cbl):
    k = pl.kernel(
        _edge_body,
        out_type=jax.ShapeDtypeStruct((NC, AGG_PAD, D), jnp.float32),
        mesh=_mesh(),
        scratch_types=[
            pltpu.VMEM((2, EB, D), jnp.float32),
            pltpu.VMEM((CB * D,), jnp.float32),
            pltpu.VMEM((2, 8, EB), jnp.int32),
            pltpu.VMEM((2, EB), jnp.int32),
            pltpu.VMEM_SHARED((AGG_PAD, D), jnp.float32),
            pltpu.SemaphoreType.DMA,
            pltpu.SemaphoreType.DMA,
            pltpu.SemaphoreType.DMA,
            pltpu.SemaphoreType.DMA,
            pltpu.SemaphoreType.DMA,
            pltpu.SemaphoreType.DMA,
        ],
    )
    return k(h, idx3, cbl)


def _edge_body(h_hbm, idx_hbm, cb_hbm, out_hbm,
               rows_v, cb_v, idx_v, di_v, agg_sh,
               isem0, isem1, gsem0, gsem1, ssem0, ssem1):
    c = lax.axis_index("c")
    s = lax.axis_index("s")
    wid = c * NS + s
    isem = (isem0, isem1)
    gsem = (gsem0, gsem1)
    ssem = (ssem0, ssem1)

    pltpu.sync_copy(cb_hbm, cb_v)

    def zrow(j, carry):
        for g in range(D // 16):
            rows_v[0, j, pl.ds(g * 16, 16)] = jnp.zeros((16,), jnp.float32)
        return carry

    lax.fori_loop(0, EB, zrow, 0)
    tbase = s * ROWS_T
    for j in range(ROWS_T // EB):
        pltpu.sync_copy(rows_v.at[0], agg_sh.at[pl.ds(tbase + j * EB, EB)])
    rem = ROWS_T - (ROWS_T // EB) * EB
    if rem:
        pltpu.sync_copy(rows_v.at[0, pl.ds(0, rem)],
                        agg_sh.at[pl.ds(tbase + (ROWS_T // EB) * EB, rem)])
    plsc.subcore_barrier()

    bbase = wid * NBLK_W

    def fire_idx(kk, b):
        pltpu.async_copy(idx_hbm.at[bbase + kk], idx_v.at[b], isem[b])

    def drain_idx(b):
        pltpu.make_async_copy(idx_hbm.at[0], idx_v.at[b], isem[b]).wait()

    def fire_gathers(b):
        pltpu.async_copy(h_hbm.at[idx_v.at[b, 0]], rows_v.at[b], gsem[b])

    def drain_gathers(b):
        pltpu.make_async_copy(h_hbm.at[idx_v.at[b, 0]], rows_v.at[b],
                              gsem[b]).wait()

    def fire_scatter(b):
        pltpu.async_copy(rows_v.at[b], agg_sh.at[di_v.at[b]], ssem[b],
                         add=True)

    def drain_scatter(b):
        pltpu.make_async_copy(rows_v.at[b], agg_sh.at[di_v.at[b]],
                              ssem[b]).wait()

    fire_idx(0, 0)
    fire_idx(1, 1)
    drain_idx(0)
    fire_gathers(0)

    def superstep(ks, carry):
        for b in range(2):
            kk = 2 * ks + b
            drain_gathers(b)
            for jj in range(EB // 16):
                di_v[b, pl.ds(jj * 16, 16)] = idx_v[b, 2, pl.ds(jj * 16, 16)]

            @plsc.parallel_loop(0, EB // 16, unroll=2)
            def _egroup(j):
                civ = idx_v[b, 1, pl.ds(j * 16, 16)] * D
                for i in range(16):
                    e = j * 16 + i
                    rowb = civ[i]
                    hs = [rows_v[b, e, pl.ds(g * 16, 16)]
                          for g in range(D // 16)]
                    es = [cb_v[pl.ds(rowb + g * 16, 16)]
                          for g in range(D // 16)]
                    for g in range(D // 16):
                        rows_v[b, e, pl.ds(g * 16, 16)] = jnp.maximum(
                            hs[g] + es[g], 0.0)

            fire_scatter(b)

            @pl.when(kk >= 1)
            def _():
                drain_scatter(1 - b)

            @pl.when(kk + 2 < NBLK_W)
            def _():
                fire_idx(kk + 2, b)

            @pl.when(kk + 1 < NBLK_W)
            def _():
                drain_idx(1 - b)
                fire_gathers(1 - b)

        return carry

    lax.fori_loop(0, NBLK_W // 2, superstep, 0)
    drain_scatter(1)
    plsc.subcore_barrier()
    pltpu.sync_copy(agg_sh.at[pl.ds(tbase, ROWS_T)],
                    out_hbm.at[c, pl.ds(tbase, ROWS_T)])


BR = 1000
NB = N // BR


def _tc1_body(epsm_ref, h_ref, a0_ref, a1_ref, w_ref, b_ref,
              y_ref, st_ref, acc_ref):
    t = epsm_ref[0, 0] * h_ref[...] + a0_ref[...] + a1_ref[...]
    y = jnp.dot(t, w_ref[...], preferred_element_type=jnp.float32) + b_ref[...]
    y_ref[...] = y
    i = pl.program_id(0)
    contrib = jnp.stack([jnp.sum(y, axis=0), jnp.sum(y * y, axis=0)])
    prev = jnp.where(i == 0, jnp.zeros_like(contrib), acc_ref[...])
    acc_ref[...] = prev + contrib

    @pl.when(i == NB - 1)
    def _():
        st_ref[...] = acc_ref[...]


def _tc2_body(y1_ref, st_ref, g_ref, bb_ref, w_ref, b_ref,
              y_ref, st2_ref, acc_ref):
    m = st_ref[0] * (1.0 / N)
    v = st_ref[1] * (1.0 / N) - m * m
    z = (y1_ref[...] - m) * lax.rsqrt(v + 1e-5) * g_ref[...] + bb_ref[...]
    z = jnp.maximum(z, 0.0)
    y = jnp.dot(z, w_ref[...], preferred_element_type=jnp.float32) + b_ref[...]
    y_ref[...] = y
    i = pl.program_id(0)
    contrib = jnp.stack([jnp.sum(y, axis=0), jnp.sum(y * y, axis=0)])
    prev = jnp.where(i == 0, jnp.zeros_like(contrib), acc_ref[...])
    acc_ref[...] = prev + contrib

    @pl.when(i == NB - 1)
    def _():
        st2_ref[...] = acc_ref[...]


def _tc3_body(relu, y2_ref, st_ref, g_ref, bb_ref, h_ref):
    m = st_ref[0] * (1.0 / N)
    v = st_ref[1] * (1.0 / N) - m * m
    h = (y2_ref[...] - m) * lax.rsqrt(v + 1e-5) * g_ref[...] + bb_ref[...]
    if relu:
        h = jnp.maximum(h, 0.0)
    h_ref[...] = h


def _row_spec(cols):
    return pl.BlockSpec((BR, cols), lambda i: (i, 0))


def _full_spec(r, cols):
    return pl.BlockSpec((r, cols), lambda i: (0, 0))


def _tc1(epsm, h, a0, a1, w1, b1):
    return pl.pallas_call(
        _tc1_body,
        grid=(NB,),
        in_specs=[
            pl.BlockSpec(memory_space=pltpu.SMEM),
            _row_spec(D), _row_spec(D), _row_spec(D),
            _full_spec(D, 2 * D), _full_spec(1, 2 * D),
        ],
        out_specs=[_row_spec(2 * D), _full_spec(2, 2 * D)],
        out_shape=[jax.ShapeDtypeStruct((N, 2 * D), jnp.float32),
                   jax.ShapeDtypeStruct((2, 2 * D), jnp.float32)],
        scratch_shapes=[pltpu.VMEM((2, 2 * D), jnp.float32)],
    )(epsm, h, a0, a1, w1, b1)


def _tc2(y1, st1, g1, bb1, w2, b2):
    return pl.pallas_call(
        _tc2_body,
        grid=(NB,),
        in_specs=[
            _row_spec(2 * D), _full_spec(2, 2 * D),
            _full_spec(1, 2 * D), _full_spec(1, 2 * D),
            _full_spec(2 * D, D), _full_spec(1, D),
        ],
        out_specs=[_row_spec(D), _full_spec(2, D)],
        out_shape=[jax.ShapeDtypeStruct((N, D), jnp.float32),
                   jax.ShapeDtypeStruct((2, D), jnp.float32)],
        scratch_shapes=[pltpu.VMEM((2, D), jnp.float32)],
    )(y1, st1, g1, bb1, w2, b2)


def _tc3(y2, st2, g, bb, relu):
    return pl.pallas_call(
        functools.partial(_tc3_body, relu),
        grid=(NB,),
        in_specs=[
            _row_spec(D), _full_spec(2, D),
            _full_spec(1, D), _full_spec(1, D),
        ],
        out_specs=_row_spec(D),
        out_shape=jax.ShapeDtypeStruct((N, D), jnp.float32),
    )(y2, st2, g, bb)


def kernel(x, edge_index, edge_attr, atom_tables, bond_tables,
           W1, b1, bn1_g, bn1_b, W2, b2, eps, obn_g, obn_b):
    x = x.astype(jnp.int32)
    edge_index = edge_index.astype(jnp.int32)
    edge_attr = edge_attr.astype(jnp.int32)

    ax = x.T + (jnp.arange(NF, dtype=jnp.int32) * AV)[:, None]
    ax_pad = jnp.pad(ax, ((0, 0), (0, N_PAD - N))).reshape(NF * N_PAD)
    tbl = atom_tables.reshape(NF * AV, D)

    bt = bond_tables
    cb = (bt[:, 0][:, :, None, None, :] + bt[:, 1][:, None, :, None, :]
          + bt[:, 2][:, None, None, :, :]).reshape(L, CB * D)

    src = edge_index[0]
    dst = edge_index[1]
    ci = (edge_attr[:, 0] * BV + edge_attr[:, 1]) * BV + edge_attr[:, 2]

    npad = E_PAD - E
    sp = jnp.concatenate([src, jnp.zeros((npad,), jnp.int32)]).reshape(
        NBLK, 1, EB)
    cp = jnp.concatenate([ci, jnp.zeros((npad,), jnp.int32)]).reshape(
        NBLK, 1, EB)
    dp = jnp.concatenate([dst, jnp.full((npad,), DUMMY_DST, jnp.int32)]
                         ).reshape(NBLK, 1, EB)
    idx3 = jnp.concatenate(
        [sp, cp, dp, jnp.zeros((NBLK, 5, EB), jnp.int32)], axis=1)

    h = _atom_kernel(ax_pad, tbl)[:N]
    for layer in range(L):
        agg2 = _edge_kernel(h, idx3, cb[layer])
        epsm = (1.0 + eps[layer]).reshape(1, 1)
        y1, st1 = _tc1(epsm, h, agg2[0, :N], agg2[1, :N], W1[layer],
                       b1[layer].reshape(1, 2 * D))
        y2, st2 = _tc2(y1, st1, bn1_g[layer].reshape(1, 2 * D),
                       bn1_b[layer].reshape(1, 2 * D), W2[layer],
                       b2[layer].reshape(1, D))
        h = _tc3(y2, st2, obn_g[layer].reshape(1, D),
                 obn_b[layer].reshape(1, D), relu=(layer < L - 1))
    return h


# speedup vs baseline: 6.4617x; 1.1219x over previous
import functools

import jax
import jax.numpy as jnp
from jax import lax
from jax.experimental import pallas as pl
from jax.experimental.pallas import tpu as pltpu
from jax.experimental.pallas import tpu_sc as plsc

N = 10000
E = 320000
D = 128
L = 3
NF = 9
BF = 3
AV = 120
BV = 6

NC = 2
NS = 16
NW = NC * NS

N_PAD = 10240
NODES_W = N_PAD // NW
PCH = 80
N_CHUNKS = NODES_W // PCH

SUB = 80
EB = 160
NBLK_W = 63
E_W_PAD = NBLK_W * EB
E_PAD = E_W_PAD * NW
NBLK = E_PAD // EB
IDXB = 6 * SUB
AGG_PAD = 10112
ROWS_T = AGG_PAD // NS
DUMMY_DST = AGG_PAD - 8

CB = BV * BV * BV

def _wid():
    return lax.axis_index("c") * NS + lax.axis_index("s")


def _mesh():
    return plsc.VectorSubcoreMesh(core_axis_name="c", subcore_axis_name="s",
                                  num_cores=NC, num_subcores=NS)


def _atom_kernel(ax_pad, tbl):
    k = pl.kernel(
        _atom_body,
        out_type=jax.ShapeDtypeStruct((N_PAD, D), jnp.float32),
        mesh=_mesh(),
        scratch_types=[
            pltpu.VMEM((NF, PCH), jnp.int32),
            pltpu.VMEM((NF, PCH, D), jnp.float32),
            pltpu.VMEM((PCH, D), jnp.float32),
            pltpu.SemaphoreType.DMA,
        ],
    )
    return k(ax_pad, tbl)


def _atom_body(ax_hbm, tbl_hbm, out_hbm, idx_v, rows_v, out_v, sem):
    wid = _wid()

    def chunk(k, carry):
        nb = wid * NODES_W + k * PCH
        for f in range(NF):
            pltpu.sync_copy(ax_hbm.at[pl.ds(f * N_PAD + nb, PCH)], idx_v.at[f])
        cps = [pltpu.async_copy(tbl_hbm.at[idx_v.at[f]], rows_v.at[f], sem)
               for f in range(NF)]
        for cp in cps:
            cp.wait()

        def node(e, carry2):
            for g in range(D // 16):
                sl = pl.ds(g * 16, 16)
                acc = rows_v[0, e, sl]
                for f in range(1, NF):
                    acc = acc + rows_v[f, e, sl]
                out_v[e, sl] = acc
            return carry2

        lax.fori_loop(0, PCH, node, 0)
        pltpu.sync_copy(out_v, out_hbm.at[pl.ds(nb, PCH)])
        return carry

    lax.fori_loop(0, N_CHUNKS, chunk, 0)


def _edge_kernel(h, idx_all, cip, cbl):
    k = pl.kernel(
        _edge_body,
        out_type=jax.ShapeDtypeStruct((NC, AGG_PAD, D), jnp.float32),
        mesh=_mesh(),
        scratch_types=[
            pltpu.VMEM((CB * D,), jnp.float32),
            pltpu.VMEM((IDXB,), jnp.int32),
            pltpu.VMEM((2, SUB), jnp.int32),
            pltpu.VMEM((EB, D), jnp.float32),
            pltpu.VMEM_SHARED((AGG_PAD, D), jnp.float32),
            pltpu.SemaphoreType.DMA,
            pltpu.SemaphoreType.DMA,
        ],
    )
    return k(h, idx_all, cip, cbl)


def _edge_body(h_hbm, idx_hbm, ci_hbm, cb_hbm, out_hbm,
               cb_v, idx_v, di2_v, rows_v, agg_sh, sem, isem):
    c = lax.axis_index("c")
    s = lax.axis_index("s")
    wid = c * NS + s

    pltpu.sync_copy(cb_hbm, cb_v)

    def zrow(j, carry):
        for g in range(D // 16):
            rows_v[j, pl.ds(g * 16, 16)] = jnp.zeros((16,), jnp.float32)
        return carry

    lax.fori_loop(0, EB, zrow, 0)
    tbase = s * ROWS_T
    for j in range(ROWS_T // EB):
        pltpu.sync_copy(rows_v, agg_sh.at[pl.ds(tbase + j * EB, EB)])
    rem = ROWS_T - (ROWS_T // EB) * EB
    if rem:
        pltpu.sync_copy(rows_v.at[pl.ds(0, rem)],
                        agg_sh.at[pl.ds(tbase + (ROWS_T // EB) * EB, rem)])
    plsc.subcore_barrier()

    pltpu.sync_copy(idx_hbm.at[pl.ds(wid * NBLK_W * IDXB, IDXB)], idx_v)

    def block(k, carry):
        g0 = pltpu.async_copy(h_hbm.at[idx_v.at[pl.ds(0, SUB)]],
                              rows_v.at[pl.ds(0, SUB)], sem)
        g1 = pltpu.async_copy(h_hbm.at[idx_v.at[pl.ds(SUB, SUB)]],
                              rows_v.at[pl.ds(SUB, SUB)], sem)
        for jj in range(EB // 16):
            di2_v[jj // (SUB // 16), pl.ds((jj % (SUB // 16)) * 16, 16)] = (
                idx_v[pl.ds(4 * SUB + jj * 16, 16)])
        g0.wait()
        g1.wait()

        @plsc.parallel_loop(0, EB // 16, unroll=4)
        def _egroup(j):
            civ = idx_v[pl.ds(2 * SUB + j * 16, 16)] * D
            for i in range(16):
                e = j * 16 + i
                rowb = civ[i]
                hs = [rows_v[e, pl.ds(g * 16, 16)] for g in range(D // 16)]
                es = [cb_v[pl.ds(rowb + g * 16, 16)] for g in range(D // 16)]
                for g in range(D // 16):
                    rows_v[e, pl.ds(g * 16, 16)] = jnp.maximum(
                        hs[g] + es[g], 0.0)
        @pl.when(k + 1 < NBLK_W)
        def _():
            off = (wid * NBLK_W + k + 1) * IDXB
            pltpu.async_copy(idx_hbm.at[pl.ds(off, IDXB)], idx_v, isem)

        s0 = pltpu.async_copy(rows_v.at[pl.ds(0, SUB)],
                              agg_sh.at[di2_v.at[0]], sem, add=True)
        s1 = pltpu.async_copy(rows_v.at[pl.ds(SUB, SUB)],
                              agg_sh.at[di2_v.at[1]], sem, add=True)
        s0.wait()
        s1.wait()

        @pl.when(k + 1 < NBLK_W)
        def _():
            pltpu.make_async_copy(idx_hbm.at[pl.ds(0, IDXB)], idx_v,
                                  isem).wait()

        return carry

    lax.fori_loop(0, NBLK_W, block, 0)
    plsc.subcore_barrier()
    pltpu.sync_copy(agg_sh.at[pl.ds(tbase, ROWS_T)],
                    out_hbm.at[c, pl.ds(tbase, ROWS_T)])


BR = 1000
NB = N // BR


def _tc1_body(epsm_ref, h_ref, a0_ref, a1_ref, w_ref, b_ref,
              y_ref, st_ref, acc_ref):
    t = epsm_ref[0, 0] * h_ref[...] + a0_ref[...] + a1_ref[...]
    y = jnp.dot(t, w_ref[...], preferred_element_type=jnp.float32) + b_ref[...]
    y_ref[...] = y
    i = pl.program_id(0)
    contrib = jnp.stack([jnp.sum(y, axis=0), jnp.sum(y * y, axis=0)])
    prev = jnp.where(i == 0, jnp.zeros_like(contrib), acc_ref[...])
    acc_ref[...] = prev + contrib

    @pl.when(i == NB - 1)
    def _():
        st_ref[...] = acc_ref[...]


def _tc2_body(y1_ref, st_ref, g_ref, bb_ref, w_ref, b_ref,
              y_ref, st2_ref, acc_ref):
    m = st_ref[0] * (1.0 / N)
    v = st_ref[1] * (1.0 / N) - m * m
    z = (y1_ref[...] - m) * lax.rsqrt(v + 1e-5) * g_ref[...] + bb_ref[...]
    z = jnp.maximum(z, 0.0)
    y = jnp.dot(z, w_ref[...], preferred_element_type=jnp.float32) + b_ref[...]
    y_ref[...] = y
    i = pl.program_id(0)
    contrib = jnp.stack([jnp.sum(y, axis=0), jnp.sum(y * y, axis=0)])
    prev = jnp.where(i == 0, jnp.zeros_like(contrib), acc_ref[...])
    acc_ref[...] = prev + contrib

    @pl.when(i == NB - 1)
    def _():
        st2_ref[...] = acc_ref[...]


def _tc3_body(relu, y2_ref, st_ref, g_ref, bb_ref, h_ref):
    m = st_ref[0] * (1.0 / N)
    v = st_ref[1] * (1.0 / N) - m * m
    h = (y2_ref[...] - m) * lax.rsqrt(v + 1e-5) * g_ref[...] + bb_ref[...]
    if relu:
        h = jnp.maximum(h, 0.0)
    h_ref[...] = h


def _row_spec(cols):
    return pl.BlockSpec((BR, cols), lambda i: (i, 0))


def _full_spec(r, cols):
    return pl.BlockSpec((r, cols), lambda i: (0, 0))


def _tc1(epsm, h, a0, a1, w1, b1):
    return pl.pallas_call(
        _tc1_body,
        grid=(NB,),
        in_specs=[
            pl.BlockSpec(memory_space=pltpu.SMEM),
            _row_spec(D), _row_spec(D), _row_spec(D),
            _full_spec(D, 2 * D), _full_spec(1, 2 * D),
        ],
        out_specs=[_row_spec(2 * D), _full_spec(2, 2 * D)],
        out_shape=[jax.ShapeDtypeStruct((N, 2 * D), jnp.float32),
                   jax.ShapeDtypeStruct((2, 2 * D), jnp.float32)],
        scratch_shapes=[pltpu.VMEM((2, 2 * D), jnp.float32)],
    )(epsm, h, a0, a1, w1, b1)


def _tc2(y1, st1, g1, bb1, w2, b2):
    return pl.pallas_call(
        _tc2_body,
        grid=(NB,),
        in_specs=[
            _row_spec(2 * D), _full_spec(2, 2 * D),
            _full_spec(1, 2 * D), _full_spec(1, 2 * D),
            _full_spec(2 * D, D), _full_spec(1, D),
        ],
        out_specs=[_row_spec(D), _full_spec(2, D)],
        out_shape=[jax.ShapeDtypeStruct((N, D), jnp.float32),
                   jax.ShapeDtypeStruct((2, D), jnp.float32)],
        scratch_shapes=[pltpu.VMEM((2, D), jnp.float32)],
    )(y1, st1, g1, bb1, w2, b2)


def _tc3(y2, st2, g, bb, relu):
    return pl.pallas_call(
        functools.partial(_tc3_body, relu),
        grid=(NB,),
        in_specs=[
            _row_spec(D), _full_spec(2, D),
            _full_spec(1, D), _full_spec(1, D),
        ],
        out_specs=_row_spec(D),
        out_shape=jax.ShapeDtypeStruct((N, D), jnp.float32),
    )(y2, st2, g, bb)


def kernel(x, edge_index, edge_attr, atom_tables, bond_tables,
           W1, b1, bn1_g, bn1_b, W2, b2, eps, obn_g, obn_b):
    x = x.astype(jnp.int32)
    edge_index = edge_index.astype(jnp.int32)
    edge_attr = edge_attr.astype(jnp.int32)

    ax = x.T + (jnp.arange(NF, dtype=jnp.int32) * AV)[:, None]
    ax_pad = jnp.pad(ax, ((0, 0), (0, N_PAD - N))).reshape(NF * N_PAD)
    tbl = atom_tables.reshape(NF * AV, D)

    bt = bond_tables
    cb = (bt[:, 0][:, :, None, None, :] + bt[:, 1][:, None, :, None, :]
          + bt[:, 2][:, None, None, :, :]).reshape(L, CB * D)

    src = edge_index[0]
    dst = edge_index[1]
    ci = (edge_attr[:, 0] * BV + edge_attr[:, 1]) * BV + edge_attr[:, 2]

    npad = E_PAD - E
    sp = jnp.concatenate([src, jnp.zeros((npad,), jnp.int32)]).reshape(
        NBLK, 2, SUB)
    cp = jnp.concatenate([ci, jnp.zeros((npad,), jnp.int32)]).reshape(
        NBLK, 2, SUB)
    dp = jnp.concatenate([dst, jnp.full((npad,), DUMMY_DST, jnp.int32)]
                         ).reshape(NBLK, 2, SUB)
    idx_all = jnp.concatenate([sp, cp, dp], axis=1).reshape(NBLK * IDXB)

    cip = cp.reshape(E_PAD)

    h = _atom_kernel(ax_pad, tbl)[:N]
    for layer in range(L):
        agg2 = _edge_kernel(h, idx_all, cip, cb[layer])
        epsm = (1.0 + eps[layer]).reshape(1, 1)
        y1, st1 = _tc1(epsm, h, agg2[0, :N], agg2[1, :N], W1[layer],
                       b1[layer].reshape(1, 2 * D))
        y2, st2 = _tc2(y1, st1, bn1_g[layer].reshape(1, 2 * D),
                       bn1_b[layer].reshape(1, 2 * D), W2[layer],
                       b2[layer].reshape(1, D))
        h = _tc3(y2, st2, obn_g[layer].reshape(1, D),
                 obn_b[layer].reshape(1, D), relu=(layer < L - 1))
    return h
